# Initial kernel scaffold; baseline (speedup 1.0000x reference)
import jax
import jax.numpy as jnp
from jax.experimental import pallas as pl


def kernel(doc_features, word_features, edge_index, edge_weight, mask_idx, y, lin_w, lin_b, conv1_w, conv1_b, conv2_w, conv2_b, cls1_w, cls1_b, cls2_w, cls2_b):
    raise NotImplementedError("write your pallas kernel here")


# speedup vs baseline: 3.8032x; 3.8032x over previous
import functools

import jax
import jax.numpy as jnp
from jax import lax
from jax.experimental import pallas as pl
from jax.experimental.pallas import tpu as pltpu
from jax.experimental.pallas import tpu_sc as plsc

NC, NS, LANES = 2, 16, 16
E = 160000
N = 10000
NPAD = 10240
PT = NPAD // NS
CH = 80
EPT = E // NS
NCHUNK = EPT // CH
CHE = 40
W1H = 128
W2F = 128
NMASK = 1000
MROWS = 1024

MB = 1024


@functools.cache
def _sc_mesh(num_cores):
    return plsc.VectorSubcoreMesh(
        core_axis_name="c", subcore_axis_name="s", num_cores=num_cores)


_sc_params = pltpu.CompilerParams(needs_layout_passes=False)


def _norm_body(src_h, dst_h, ew_h, nrm_h, iv_h,
               sidx, didx, vals, nbuf, disb, ivb, dis_tbl, deg_sp, dis_sp):
    t = lax.axis_index("s")
    zero = jnp.zeros((LANES,), jnp.float32)

    def z_body(i, _):
        disb[pl.ds(i * LANES, LANES)] = zero
        return 0
    lax.fori_loop(0, PT // LANES, z_body, 0)
    pltpu.sync_copy(disb, deg_sp.at[pl.ds(t * PT, PT)])
    plsc.subcore_barrier()

    def deg_body(k, _):
        off = t * EPT + k * CH
        pltpu.sync_copy(dst_h.at[pl.ds(off, CH)], didx)
        pltpu.sync_copy(ew_h.at[pl.ds(off, CH)], vals)
        pltpu.sync_copy(vals, deg_sp.at[didx], add=True)
        return 0
    lax.fori_loop(0, NCHUNK, deg_body, 0)
    plsc.subcore_barrier()

    pltpu.sync_copy(deg_sp.at[pl.ds(t * PT, PT)], disb)

    def dis_body(i, _):
        sl = pl.ds(i * LANES, LANES)
        x = disb[sl] + 1.0
        bits = lax.bitcast_convert_type(x, jnp.int32)
        y = lax.bitcast_convert_type(jnp.int32(0x5F3759DF) - (bits >> 1),
                                     jnp.float32)
        hx = 0.5 * x
        y = y * (1.5 - hx * y * y)
        y = y * (1.5 - hx * y * y)
        y = y * (1.5 - hx * y * y)
        disb[sl] = y
        ivb[sl] = y * y
        return 0
    lax.fori_loop(0, PT // LANES, dis_body, 0)
    pltpu.sync_copy(disb, dis_sp.at[pl.ds(t * PT, PT)])
    pltpu.sync_copy(ivb, iv_h.at[pl.ds(t * PT, PT)])
    plsc.subcore_barrier()

    pltpu.sync_copy(dis_sp, dis_tbl)

    def nrm_body(k, _):
        off = t * EPT + k * CH
        pltpu.sync_copy(src_h.at[pl.ds(off, CH)], sidx)
        pltpu.sync_copy(dst_h.at[pl.ds(off, CH)], didx)
        pltpu.sync_copy(ew_h.at[pl.ds(off, CH)], vals)
        for g in range(CH // LANES):
            sl = pl.ds(g * LANES, LANES)
            ds_ = plsc.load_gather(dis_tbl, [sidx[sl]])
            dd = plsc.load_gather(dis_tbl, [didx[sl]])
            nbuf[sl] = ds_ * vals[sl] * dd
        pltpu.sync_copy(nbuf, nrm_h.at[pl.ds(off, CH)])
        return 0
    lax.fori_loop(0, NCHUNK, nrm_body, 0)


@functools.cache
def _norm_call():
    return pl.kernel(
        _norm_body,
        out_type=[jax.ShapeDtypeStruct((E,), jnp.float32),
                  jax.ShapeDtypeStruct((NPAD,), jnp.float32)],
        mesh=_sc_mesh(1),
        compiler_params=_sc_params,
        scratch_types=[
            pltpu.VMEM((CH,), jnp.int32),
            pltpu.VMEM((CH,), jnp.int32),
            pltpu.VMEM((CH,), jnp.float32),
            pltpu.VMEM((CH,), jnp.float32),
            pltpu.VMEM((PT,), jnp.float32),
            pltpu.VMEM((PT,), jnp.float32),
            pltpu.VMEM((NPAD,), jnp.float32),
            pltpu.VMEM_SHARED((NPAD,), jnp.float32),
            pltpu.VMEM_SHARED((NPAD,), jnp.float32),
        ],
    )


def _agg_body(split_width, xw_h, src_h, dst_h, nrm_h, out_h,
              sidx, didx, nrm, rows, zbuf, acc, sem):
    c = lax.axis_index("c")
    t = lax.axis_index("s")
    zero = jnp.zeros((LANES,), jnp.float32)

    def zb_body(i, _):
        for q in range(W1H // LANES):
            zbuf[i, pl.ds(q * LANES, LANES)] = zero
        return 0
    lax.fori_loop(0, 128, zb_body, 0)
    for i in range(PT // 128):
        pltpu.sync_copy(zbuf, acc.at[pl.ds(t * PT + i * 128, 128)])
    plsc.subcore_barrier()

    if split_width:
        ebase = t * EPT
        nch = EPT // CHE
    else:
        ebase = (c * NS + t) * (E // 32)
        nch = (E // 32) // CHE

    def chunk(k, _):
        off = ebase + k * CHE
        pltpu.sync_copy(src_h.at[pl.ds(off, CHE)], sidx)
        pltpu.sync_copy(dst_h.at[pl.ds(off, CHE)], didx)
        pltpu.sync_copy(nrm_h.at[pl.ds(off, CHE)], nrm)
        if split_width:
            pltpu.async_copy(xw_h.at[c].at[sidx], rows, sem).wait()
        else:
            pltpu.async_copy(xw_h.at[sidx], rows, sem).wait()

        def srow(j, _):
            spl = plsc.load_gather(nrm, [jnp.zeros((LANES,), jnp.int32) + j])
            for q in range(W1H // LANES):
                sl = pl.ds(q * LANES, LANES)
                rows[j, sl] = rows[j, sl] * spl
            return 0
        lax.fori_loop(0, CHE, srow, 0)
        pltpu.sync_copy(rows, acc.at[didx], add=True)
        return 0
    lax.fori_loop(0, nch, chunk, 0)
    plsc.subcore_barrier()
    sl = pl.ds(t * PT, PT)
    pltpu.sync_copy(acc.at[sl], out_h.at[c].at[sl])


@functools.cache
def _agg_call(split_width):
    return pl.kernel(
        functools.partial(_agg_body, split_width),
        out_type=jax.ShapeDtypeStruct((NC, NPAD, W1H), jnp.float32),
        mesh=_sc_mesh(2),
        compiler_params=_sc_params,
        scratch_types=[
            pltpu.VMEM((CHE,), jnp.int32),
            pltpu.VMEM((CHE,), jnp.int32),
            pltpu.VMEM((CHE,), jnp.float32),
            pltpu.VMEM((CHE, W1H), jnp.float32),
            pltpu.VMEM((128, W1H), jnp.float32),
            pltpu.VMEM_SHARED((NPAD, W1H), jnp.float32),
            pltpu.SemaphoreType.DMA,
        ],
    )


def _mask_body(p2_h, xw2_h, iv_h, y_h, midx_h,
               p2m_h, xw2m_h, ivm_h, ym_h, ib, rb, ivb, yb, sem):
    c = lax.axis_index("c")
    s = lax.axis_index("s")
    nm = MROWS // 32
    base = (s * NC + c) * nm
    pltpu.sync_copy(midx_h.at[pl.ds(base, nm)], ib)
    for h in range(NC):
        pltpu.async_copy(p2_h.at[h].at[ib], rb, sem).wait()
        pltpu.sync_copy(rb, p2m_h.at[h].at[pl.ds(base, nm)])
    pltpu.async_copy(xw2_h.at[ib], rb, sem).wait()
    pltpu.sync_copy(rb, xw2m_h.at[pl.ds(base, nm)])
    pltpu.async_copy(iv_h.at[ib], ivb, sem).wait()
    pltpu.sync_copy(ivb, ivm_h.at[pl.ds(base, nm)])
    pltpu.async_copy(y_h.at[ib], yb, sem).wait()
    pltpu.sync_copy(yb, ym_h.at[pl.ds(base, nm)])


@functools.cache
def _mask_call():
    return pl.kernel(
        _mask_body,
        out_type=[jax.ShapeDtypeStruct((NC, MROWS, W2F), jnp.float32),
                  jax.ShapeDtypeStruct((MROWS, W2F), jnp.float32),
                  jax.ShapeDtypeStruct((MROWS,), jnp.float32),
                  jax.ShapeDtypeStruct((MROWS,), jnp.int32)],
        mesh=_sc_mesh(2),
        compiler_params=_sc_params,
        scratch_types=[
            pltpu.VMEM((MROWS // 32,), jnp.int32),
            pltpu.VMEM((MROWS // 32, W2F), jnp.float32),
            pltpu.VMEM((MROWS // 32,), jnp.float32),
            pltpu.VMEM((MROWS // 32,), jnp.int32),
            pltpu.SemaphoreType.DMA,
        ],
    )


def _lin_block(x_ref, w_ref, b_ref, o_ref):
    o_ref[...] = jnp.dot(x_ref[...], w_ref[...],
                         preferred_element_type=jnp.float32) + b_ref[...]


_lin_call = pl.pallas_call(
    _lin_block,
    grid=(8,),
    in_specs=[pl.BlockSpec((1000, 300), lambda m: (m, 0)),
              pl.BlockSpec((300, 768), lambda m: (0, 0)),
              pl.BlockSpec((1, 768), lambda m: (0, 0))],
    out_specs=pl.BlockSpec((1000, 768), lambda m: (m, 0)),
    out_shape=jax.ShapeDtypeStruct((8000, 768), jnp.float32),
)


def _xw1_block(x_ref, w_ref, o_ref):
    o_ref[0] = jnp.dot(x_ref[...], w_ref[0],
                       preferred_element_type=jnp.float32)


_xw1_call = pl.pallas_call(
    _xw1_block,
    grid=(NPAD // MB, 2),
    in_specs=[pl.BlockSpec((MB, 768), lambda m, h: (m, 0)),
              pl.BlockSpec((1, 768, W1H), lambda m, h: (h, 0, 0))],
    out_specs=pl.BlockSpec((1, MB, W1H), lambda m, h: (h, m, 0)),
    out_shape=jax.ShapeDtypeStruct((NC, NPAD, W1H), jnp.float32),
)


def _mid_block(p_ref, xw_ref, iv_ref, b1_ref, w2_ref, o_ref):
    iv = iv_ref[...]
    x1a = jnp.maximum(p_ref[0] + iv * xw_ref[0] + b1_ref[0], 0.0)
    x1b = jnp.maximum(p_ref[1] + iv * xw_ref[1] + b1_ref[1], 0.0)
    x1 = jnp.concatenate([x1a, x1b], axis=1)
    o_ref[...] = jnp.dot(x1, w2_ref[...], preferred_element_type=jnp.float32)


_mid_call = pl.pallas_call(
    _mid_block,
    grid=(NPAD // MB,),
    in_specs=[pl.BlockSpec((NC, MB, W1H), lambda m: (0, m, 0)),
              pl.BlockSpec((NC, MB, W1H), lambda m: (0, m, 0)),
              pl.BlockSpec((MB, 1), lambda m: (m, 0)),
              pl.BlockSpec((NC, 1, W1H), lambda m: (0, 0, 0)),
              pl.BlockSpec((2 * W1H, W2F), lambda m: (0, 0))],
    out_specs=pl.BlockSpec((MB, W2F), lambda m: (m, 0)),
    out_shape=jax.ShapeDtypeStruct((NPAD, W2F), jnp.float32),
)


def _cls_block(p_ref, xw_ref, iv_ref, b2_ref, w1_ref, b1c_ref, w2_ref,
               b2c_ref, o_ref):
    x = p_ref[0] + p_ref[1] + iv_ref[...] * xw_ref[...] + b2_ref[...]
    h = jnp.maximum(jnp.dot(x, w1_ref[...],
                            preferred_element_type=jnp.float32) + b1c_ref[...],
                    0.0)
    o_ref[...] = jnp.dot(h, w2_ref[...],
                         preferred_element_type=jnp.float32) + b2c_ref[...]


_cls_call = pl.pallas_call(
    _cls_block,
    in_specs=[pl.BlockSpec((NC, MROWS, W2F), lambda: (0, 0, 0)),
              pl.BlockSpec((MROWS, W2F), lambda: (0, 0)),
              pl.BlockSpec((MROWS, 1), lambda: (0, 0)),
              pl.BlockSpec((1, W2F), lambda: (0, 0)),
              pl.BlockSpec((W2F, 256), lambda: (0, 0)),
              pl.BlockSpec((1, 256), lambda: (0, 0)),
              pl.BlockSpec((256, 8), lambda: (0, 0)),
              pl.BlockSpec((1, 8), lambda: (0, 0))],
    out_specs=pl.BlockSpec((MROWS, 8), lambda: (0, 0)),
    out_shape=jax.ShapeDtypeStruct((MROWS, 8), jnp.float32),
)


def kernel(doc_features, word_features, edge_index, edge_weight, mask_idx, y,
           lin_w, lin_b, conv1_w, conv1_b, conv2_w, conv2_b,
           cls1_w, cls1_b, cls2_w, cls2_b):
    src = edge_index[0]
    dst = edge_index[1]

    nrm, invdeg = _norm_call()(src, dst, edge_weight)

    wf = _lin_call(word_features, lin_w, lin_b.reshape(1, -1))
    x = jnp.concatenate(
        [doc_features, wf,
         jnp.zeros((NPAD - N, doc_features.shape[1]), jnp.float32)], axis=0)

    w1p = jnp.pad(conv1_w, ((0, 0), (0, 2 * W1H - 200)))
    w1s = jnp.stack([w1p[:, :W1H], w1p[:, W1H:]])
    xw1s = _xw1_call(x, w1s)
    p1 = _agg_call(True)(xw1s, src, dst, nrm)

    iv = invdeg.reshape(NPAD, 1)
    b1s = jnp.pad(conv1_b, (0, 2 * W1H - 200)).reshape(NC, 1, W1H)
    w2p = jnp.pad(conv2_w, ((0, 2 * W1H - 200), (0, W2F - 100)))
    xw2 = _mid_call(p1, xw1s, iv, b1s, w2p)
    p2 = _agg_call(False)(xw2, src, dst, nrm)

    midx = jnp.concatenate(
        [mask_idx, jnp.zeros((MROWS - NMASK,), mask_idx.dtype)])
    p2m, xw2m, ivm, ym = _mask_call()(p2, xw2, invdeg, y, midx)

    b2p = jnp.pad(conv2_b, (0, W2F - 100)).reshape(1, W2F)
    w1c = jnp.pad(cls1_w, ((0, W2F - 100), (0, 0)))
    out = _cls_call(p2m, xw2m, ivm.reshape(MROWS, 1), b2p, w1c,
                    cls1_b.reshape(1, -1), cls2_w, cls2_b.reshape(1, -1))
    return out[:NMASK], ym[:NMASK]
